# Initial kernel scaffold; baseline (speedup 1.0000x reference)
import jax
import jax.numpy as jnp
from jax.experimental import pallas as pl


def kernel(h, edge_index, W1, b1, W2, b2):
    raise NotImplementedError("write your pallas kernel here")


# speedup vs baseline: 7.0666x; 7.0666x over previous
import functools

import jax
import jax.numpy as jnp
from jax import lax
from jax.experimental import pallas as pl
from jax.experimental.pallas import tpu as pltpu
from jax.experimental.pallas import tpu_sc as plsc

N_NODES = 10000
D = 128
E = 320000
NC = 2
NS = 16
NW = NC * NS
CHUNK = 80
E_PER_TILE = E // NW
N_CHUNKS = E_PER_TILE // CHUNK
ROWS_PER_TILE = 624
ROWS_TAIL = N_NODES - NS * ROWS_PER_TILE
MLP_BLOCK = 1000


def _sc_aggregate(h, send2, rec2):
    mesh = plsc.VectorSubcoreMesh(core_axis_name="c", subcore_axis_name="s")

    @functools.partial(
        pl.kernel,
        mesh=mesh,
        out_type=jax.ShapeDtypeStruct((NC, N_NODES, D), jnp.float32),
        scratch_types=[
            pltpu.VMEM((N_CHUNKS, CHUNK), jnp.int32),
            pltpu.VMEM((N_CHUNKS, CHUNK), jnp.int32),
            pltpu.VMEM((CHUNK, D), jnp.float32),
            pltpu.VMEM_SHARED((N_NODES, D), jnp.float32),
            pltpu.SemaphoreType.DMA,
        ],
    )
    def agg_kernel(h_hbm, send_hbm, rec_hbm, out_hbm,
                   sidx, ridx, rows, agg, sem):
        c = lax.axis_index("c")
        s = lax.axis_index("s")
        w = c * NS + s

        pltpu.sync_copy(send_hbm.at[w], sidx)
        pltpu.sync_copy(rec_hbm.at[w], ridx)

        zero = jnp.zeros((16,), jnp.float32)

        def zrow(r, carry):
            for cc in range(D // 16):
                rows[r, pl.ds(cc * 16, 16)] = zero
            return carry

        lax.fori_loop(0, CHUNK, zrow, 0)
        for kpart in range(ROWS_PER_TILE // CHUNK):
            pltpu.sync_copy(
                rows, agg.at[pl.ds(s * ROWS_PER_TILE + kpart * CHUNK, CHUNK)])
        zrem = ROWS_PER_TILE % CHUNK
        pltpu.sync_copy(
            rows.at[pl.ds(0, zrem)],
            agg.at[pl.ds(s * ROWS_PER_TILE + ROWS_PER_TILE - zrem, zrem)])

        @pl.when(s == NS - 1)
        def _zero_tail():
            pltpu.sync_copy(
                rows.at[pl.ds(0, ROWS_TAIL)],
                agg.at[pl.ds(NS * ROWS_PER_TILE, ROWS_TAIL)])

        plsc.subcore_barrier()

        def chunk_body(j, carry):
            pltpu.async_copy(h_hbm.at[sidx.at[j]], rows, sem).wait()
            pltpu.sync_copy(rows, agg.at[ridx.at[j]], add=True)
            return carry

        lax.fori_loop(0, N_CHUNKS, chunk_body, 0)
        plsc.subcore_barrier()

        pltpu.sync_copy(
            agg.at[pl.ds(s * ROWS_PER_TILE, ROWS_PER_TILE)],
            out_hbm.at[c, pl.ds(s * ROWS_PER_TILE, ROWS_PER_TILE)])

        @pl.when(s == NS - 1)
        def _copy_tail():
            pltpu.sync_copy(
                agg.at[pl.ds(NS * ROWS_PER_TILE, ROWS_TAIL)],
                out_hbm.at[c, pl.ds(NS * ROWS_PER_TILE, ROWS_TAIL)])

    return agg_kernel(h, send2, rec2)


def _mlp_kernel(h_ref, a_ref, w1_ref, b1_ref, w2_ref, b2_ref, o_ref):
    x = h_ref[...] + a_ref[0] + a_ref[1]
    z = lax.dot_general(
        x, w1_ref[...], dimension_numbers=(((1,), (1,)), ((), ())),
        preferred_element_type=jnp.float32,
        precision=lax.Precision.HIGHEST) + b1_ref[...]
    z = jnp.maximum(z, 0.0)
    z = lax.dot_general(
        z, w2_ref[...], dimension_numbers=(((1,), (1,)), ((), ())),
        preferred_element_type=jnp.float32,
        precision=lax.Precision.HIGHEST) + b2_ref[...]
    o_ref[...] = z


def kernel(h, edge_index, W1, b1, W2, b2):
    send2 = edge_index[0].astype(jnp.int32).reshape(NW, N_CHUNKS, CHUNK)
    rec2 = edge_index[1].astype(jnp.int32).reshape(NW, N_CHUNKS, CHUNK)
    agg = _sc_aggregate(h, send2, rec2)
    grid = N_NODES // MLP_BLOCK
    out = pl.pallas_call(
        _mlp_kernel,
        grid=(grid,),
        in_specs=[
            pl.BlockSpec((MLP_BLOCK, D), lambda i: (i, 0)),
            pl.BlockSpec((NC, MLP_BLOCK, D), lambda i: (0, i, 0)),
            pl.BlockSpec((D, D), lambda i: (0, 0)),
            pl.BlockSpec((1, D), lambda i: (0, 0)),
            pl.BlockSpec((D, D), lambda i: (0, 0)),
            pl.BlockSpec((1, D), lambda i: (0, 0)),
        ],
        out_specs=pl.BlockSpec((MLP_BLOCK, D), lambda i: (i, 0)),
        out_shape=jax.ShapeDtypeStruct((N_NODES, D), jnp.float32),
    )(h, agg, W1, b1.reshape(1, D), W2, b2.reshape(1, D))
    return out
